# baseline (speedup 1.0000x reference)
import functools

import jax
import jax.numpy as jnp
from jax import lax
from jax.experimental import pallas as pl
from jax.experimental.pallas import tpu as pltpu
from jax.experimental.pallas import tpu_sc as plsc

_R = 8
_U = 8
_BS = 512
_B_SC = 2


def _sc_kernel(B, S, D, nb):
    info = plsc.get_sparse_core_info()
    NW = info.num_cores * info.num_subcores
    L = info.num_lanes
    s_per_w = S // NW
    nblk = s_per_w // _R
    ncol = D // L
    mesh = plsc.VectorSubcoreMesh(core_axis_name="c", subcore_axis_name="s")

    @functools.partial(
        pl.kernel,
        mesh=mesh,
        out_type=jax.ShapeDtypeStruct((nb, S, D), jnp.float32),
        scratch_types=[
            pltpu.VMEM((4, _R, D), jnp.float32),
            pltpu.VMEM((2, nb, _R, D), jnp.float32),
            pltpu.VMEM((_R, D), jnp.float32),
            pltpu.SemaphoreType.DMA,
            pltpu.SemaphoreType.DMA,
            pltpu.SemaphoreType.DMA,
            pltpu.SemaphoreType.DMA,
        ],
    )
    def k(x_hbm, pe_hbm, out_hbm, pe_v, x_v, o_v, si0, si1, so0, so1):
        wid = lax.axis_index("s") * info.num_cores + lax.axis_index("c")
        s_base = wid * s_per_w
        sin = (si0, si1)
        sout = (so0, so1)

        def fire_in(blk_idx, p, q, sem):
            s0 = s_base + blk_idx * _R
            pltpu.async_copy(pe_hbm.at[pl.ds(s0, _R)], pe_v.at[q], sem)
            for b in range(nb):
                pltpu.async_copy(x_hbm.at[b, pl.ds(s0, _R)], x_v.at[p, b], sem)

        def drain_in(p, q, sem):
            pltpu.make_async_copy(pe_hbm.at[pl.ds(s_base, _R)], pe_v.at[q], sem).wait()
            for b in range(nb):
                pltpu.make_async_copy(
                    x_hbm.at[b, pl.ds(s_base, _R)], x_v.at[p, b], sem
                ).wait()

        def drain_out(q, sem):
            for b in range(nb):
                pltpu.make_async_copy(
                    pe_v.at[q], out_hbm.at[b, pl.ds(s_base, _R)], sem
                ).wait()

        def body(i, p, q, prev_fast):
            s0 = s_base + i * _R
            drain_in(p, q, sin[p])

            @pl.when(prev_fast)
            def _():
                drain_out((q + 2) % 4, sout[p])

            def scan_batch(b, acc0):
                def col(cu, acc):
                    for u in range(_U):
                        k_ = cu * _U + u
                        r, c = k_ // ncol, k_ % ncol
                        xv = x_v[p, b, r, pl.ds(c * L, L)]
                        xi = lax.bitcast_convert_type(xv, jnp.int32)
                        acc = jnp.minimum(acc, xi & jnp.int32(0x7FFFFFFF))
                    return acc

                return lax.fori_loop(0, (_R * ncol) // _U, col, acc0)

            acc = jnp.full((L,), 1, jnp.int32)
            for b in range(nb):
                acc = scan_batch(b, acc)
            block_zero = acc[0] == 0
            for l in range(1, L):
                block_zero = jnp.logical_or(block_zero, acc[l] == 0)
            fast = jnp.logical_not(block_zero)

            @pl.when(fast)
            def _():
                for b in range(nb):
                    pltpu.async_copy(
                        pe_v.at[q], out_hbm.at[b, pl.ds(s0, _R)], sout[p]
                    )

            @pl.when(block_zero)
            def _():
                for b in range(nb):
                    def row(r, carry):
                        def col(c, carry2):
                            xv = x_v[p, b, r, pl.ds(c * L, L)]
                            pv = pe_v[q, r, pl.ds(c * L, L)]
                            o_v[r, pl.ds(c * L, L)] = jnp.where(xv == 0.0, 0.0, pv)
                            return carry2

                        return lax.fori_loop(0, ncol, col, carry)

                    lax.fori_loop(0, _R, row, None)
                    pltpu.sync_copy(o_v, out_hbm.at[b, pl.ds(s0, _R)])

            return fast

        def step(j, carry):
            fA, fB = carry
            flags = [fA, fB]
            for p_ in range(4):
                i = 4 * j + p_
                p = p_ % 2
                fast = body(i, p, p_, flags[p_])
                flags.append(fast)
                if p_ < 2:
                    fire_in(i + 2, p, (p_ + 2) % 4, sin[p])
                else:

                    @pl.when(j < nblk // 4 - 1)
                    def _():
                        fire_in(i + 2, p, (p_ + 2) % 4, sin[p])

            return flags[4], flags[5]

        fire_in(0, 0, 0, sin[0])
        fire_in(1, 1, 1, sin[1])
        f = jnp.bool_(False)
        fA, fB = lax.fori_loop(0, nblk // 4, step, (f, f))

        @pl.when(fA)
        def _():
            drain_out((nblk - 2) % 4, sout[0])

        @pl.when(fB)
        def _():
            drain_out((nblk - 1) % 4, sout[1])

    return k


def _tc_body(x_ref, pe_ref, o_ref):
    o_ref[...] = jnp.where(x_ref[...] == 0.0, 0.0, pe_ref[...][None, :, :])


def _tc_call(x, pe, b_lo):
    B, S, D = x.shape
    nb = B - b_lo
    grid = (S // _BS, nb)
    return pl.pallas_call(
        _tc_body,
        grid=grid,
        in_specs=[
            pl.BlockSpec((1, _BS, D), lambda s, b: (b + b_lo, s, 0)),
            pl.BlockSpec((_BS, D), lambda s, b: (s, 0)),
        ],
        out_specs=pl.BlockSpec((1, _BS, D), lambda s, b: (b, s, 0)),
        out_shape=jax.ShapeDtypeStruct((nb, S, D), jnp.float32),
    )(x, pe)


def kernel(x, position_enc):
    B, S, D = x.shape
    pe = position_enc[:S]
    sc_out = _sc_kernel(B, S, D, _B_SC)(x, pe)
    tc_out = _tc_call(x, pe, _B_SC)
    return jnp.concatenate([sc_out, tc_out], axis=0)


# speedup vs baseline: 1.5867x; 1.5867x over previous
import functools

import jax
import jax.numpy as jnp
from jax import lax
from jax.experimental import pallas as pl
from jax.experimental.pallas import tpu as pltpu
from jax.experimental.pallas import tpu_sc as plsc

_R = 8
_U = 8
_BS = 512
_S_SC = 4096


def _sc_kernel(B, S, D, s_sc):
    info = plsc.get_sparse_core_info()
    NW = info.num_cores * info.num_subcores
    L = info.num_lanes
    nb = B
    s_per_w = s_sc // NW
    nblk = s_per_w // _R
    ncol = D // L
    mesh = plsc.VectorSubcoreMesh(core_axis_name="c", subcore_axis_name="s")

    @functools.partial(
        pl.kernel,
        mesh=mesh,
        out_type=jax.ShapeDtypeStruct((B, S, D), jnp.float32),
        scratch_types=[
            pltpu.VMEM((4, _R, D), jnp.float32),
            pltpu.VMEM((2, nb, _R, D), jnp.float32),
            pltpu.VMEM((_R, D), jnp.float32),
            pltpu.SemaphoreType.DMA,
            pltpu.SemaphoreType.DMA,
            pltpu.SemaphoreType.DMA,
            pltpu.SemaphoreType.DMA,
        ],
    )
    def k(x_hbm, pe_hbm, out_hbm, pe_v, x_v, o_v, si0, si1, so0, so1):
        wid = lax.axis_index("s") * info.num_cores + lax.axis_index("c")
        s_base = wid * s_per_w
        sin = (si0, si1)
        sout = (so0, so1)

        def fire_in(blk_idx, p, q, sem):
            s0 = s_base + blk_idx * _R
            pltpu.async_copy(pe_hbm.at[pl.ds(s0, _R)], pe_v.at[q], sem)
            for b in range(nb):
                pltpu.async_copy(x_hbm.at[b, pl.ds(s0, _R)], x_v.at[p, b], sem)

        def drain_in(p, q, sem):
            pltpu.make_async_copy(pe_hbm.at[pl.ds(s_base, _R)], pe_v.at[q], sem).wait()
            for b in range(nb):
                pltpu.make_async_copy(
                    x_hbm.at[b, pl.ds(s_base, _R)], x_v.at[p, b], sem
                ).wait()

        def drain_out(q, sem):
            for b in range(nb):
                pltpu.make_async_copy(
                    pe_v.at[q], out_hbm.at[b, pl.ds(s_base, _R)], sem
                ).wait()

        def body(i, p, q, prev_fast):
            s0 = s_base + i * _R
            drain_in(p, q, sin[p])

            @pl.when(prev_fast)
            def _():
                drain_out((q + 2) % 4, sout[p])

            def scan_batch(b, acc0):
                def col(cu, acc):
                    for u in range(_U):
                        k_ = cu * _U + u
                        r, c = k_ // ncol, k_ % ncol
                        xv = x_v[p, b, r, pl.ds(c * L, L)]
                        xi = lax.bitcast_convert_type(xv, jnp.int32)
                        acc = jnp.minimum(acc, xi & jnp.int32(0x7FFFFFFF))
                    return acc

                return lax.fori_loop(0, (_R * ncol) // _U, col, acc0)

            acc = jnp.full((L,), 1, jnp.int32)
            for b in range(nb):
                acc = scan_batch(b, acc)
            block_zero = acc[0] == 0
            for l in range(1, L):
                block_zero = jnp.logical_or(block_zero, acc[l] == 0)
            fast = jnp.logical_not(block_zero)

            @pl.when(fast)
            def _():
                for b in range(nb):
                    pltpu.async_copy(
                        pe_v.at[q], out_hbm.at[b, pl.ds(s0, _R)], sout[p]
                    )

            @pl.when(block_zero)
            def _():
                for b in range(nb):
                    def row(r, carry):
                        def col(c, carry2):
                            xv = x_v[p, b, r, pl.ds(c * L, L)]
                            pv = pe_v[q, r, pl.ds(c * L, L)]
                            o_v[r, pl.ds(c * L, L)] = jnp.where(xv == 0.0, 0.0, pv)
                            return carry2

                        return lax.fori_loop(0, ncol, col, carry)

                    lax.fori_loop(0, _R, row, None)
                    pltpu.sync_copy(o_v, out_hbm.at[b, pl.ds(s0, _R)])

            return fast

        def step(j, carry):
            fA, fB = carry
            flags = [fA, fB]
            for p_ in range(4):
                i = 4 * j + p_
                p = p_ % 2
                fast = body(i, p, p_, flags[p_])
                flags.append(fast)
                if p_ < 2:
                    fire_in(i + 2, p, (p_ + 2) % 4, sin[p])
                else:

                    @pl.when(j < nblk // 4 - 1)
                    def _():
                        fire_in(i + 2, p, (p_ + 2) % 4, sin[p])

            return flags[4], flags[5]

        fire_in(0, 0, 0, sin[0])
        fire_in(1, 1, 1, sin[1])
        f = jnp.bool_(False)
        fA, fB = lax.fori_loop(0, nblk // 4, step, (f, f))

        @pl.when(fA)
        def _():
            drain_out((nblk - 2) % 4, sout[0])

        @pl.when(fB)
        def _():
            drain_out((nblk - 1) % 4, sout[1])

    return k


def _tc_body(part_ref, x_ref, pe_ref, o_ref):
    del part_ref
    o_ref[...] = jnp.where(x_ref[...] == 0.0, 0.0, pe_ref[...][None, :, :])


def _tc_call(part, x, pe, s_lo):
    B, S, D = x.shape
    off = s_lo // _BS
    grid = ((S - s_lo) // _BS, B)
    return pl.pallas_call(
        _tc_body,
        grid=grid,
        in_specs=[
            pl.BlockSpec(memory_space=pltpu.HBM),
            pl.BlockSpec((1, _BS, D), lambda s, b: (b, s + off, 0)),
            pl.BlockSpec((_BS, D), lambda s, b: (s + off, 0)),
        ],
        out_specs=pl.BlockSpec((1, _BS, D), lambda s, b: (b, s + off, 0)),
        out_shape=jax.ShapeDtypeStruct((B, S, D), jnp.float32),
        input_output_aliases={0: 0},
    )(part, x, pe)


def kernel(x, position_enc):
    B, S, D = x.shape
    pe = position_enc[:S]
    part = _sc_kernel(B, S, D, _S_SC)(x, pe)
    return _tc_call(part, x, pe, _S_SC)


# speedup vs baseline: 1.6309x; 1.0279x over previous
import functools

import jax
import jax.numpy as jnp
from jax import lax
from jax.experimental import pallas as pl
from jax.experimental.pallas import tpu as pltpu
from jax.experimental.pallas import tpu_sc as plsc

_R = 8
_U = 8


def _sc_kernel(B, S, D):
    info = plsc.get_sparse_core_info()
    NW = info.num_cores * info.num_subcores
    L = info.num_lanes
    s_per_w = S // NW
    nblk = s_per_w // _R
    ncol = D // L
    mesh = plsc.VectorSubcoreMesh(core_axis_name="c", subcore_axis_name="s")

    @functools.partial(
        pl.kernel,
        mesh=mesh,
        out_type=jax.ShapeDtypeStruct((B, S, D), jnp.float32),
        scratch_types=[
            pltpu.VMEM((4, _R, D), jnp.float32),
            pltpu.VMEM((2, B, _R, D), jnp.float32),
            pltpu.VMEM((_R, D), jnp.float32),
            pltpu.SemaphoreType.DMA,
            pltpu.SemaphoreType.DMA,
            pltpu.SemaphoreType.DMA,
            pltpu.SemaphoreType.DMA,
            pltpu.SemaphoreType.DMA,
            pltpu.SemaphoreType.DMA,
        ],
    )
    def k(x_hbm, pe_hbm, out_hbm, pe_v, x_v, o_v, sp0, sp1, sx0, sx1, so0, so1):
        wid = lax.axis_index("s") * info.num_cores + lax.axis_index("c")
        s_base = wid * s_per_w
        spe = (sp0, sp1)
        sx = (sx0, sx1)
        sout = (so0, so1)

        def fire_in(blk_idx, p, q):
            s0 = s_base + blk_idx * _R
            pltpu.async_copy(pe_hbm.at[pl.ds(s0, _R)], pe_v.at[q], spe[p])
            for b in range(B):
                pltpu.async_copy(x_hbm.at[b, pl.ds(s0, _R)], x_v.at[p, b], sx[p])

        def drain_pe(p, q):
            pltpu.make_async_copy(
                pe_hbm.at[pl.ds(s_base, _R)], pe_v.at[q], spe[p]
            ).wait()

        def drain_x(p):
            for b in range(B):
                pltpu.make_async_copy(
                    x_hbm.at[b, pl.ds(s_base, _R)], x_v.at[p, b], sx[p]
                ).wait()

        def drain_out(q, sem):
            for b in range(B):
                pltpu.make_async_copy(
                    pe_v.at[q], out_hbm.at[b, pl.ds(s_base, _R)], sem
                ).wait()

        def body(i, p, q, prev_fast):
            s0 = s_base + i * _R
            drain_pe(p, q)

            @pl.when(prev_fast)
            def _():
                drain_out((q + 2) % 4, sout[p])

            for b in range(B):
                pltpu.async_copy(pe_v.at[q], out_hbm.at[b, pl.ds(s0, _R)], sout[p])

            drain_x(p)

            def scan_batch(b, acc0):
                def col(cu, acc):
                    for u in range(_U):
                        k_ = cu * _U + u
                        r, c = k_ // ncol, k_ % ncol
                        xv = x_v[p, b, r, pl.ds(c * L, L)]
                        xi = lax.bitcast_convert_type(xv, jnp.int32)
                        acc = jnp.minimum(acc, xi & jnp.int32(0x7FFFFFFF))
                    return acc

                return lax.fori_loop(0, (_R * ncol) // _U, col, acc0)

            acc = jnp.full((L,), 1, jnp.int32)
            for b in range(B):
                acc = scan_batch(b, acc)
            block_zero = acc[0] == 0
            for l in range(1, L):
                block_zero = jnp.logical_or(block_zero, acc[l] == 0)
            fast = jnp.logical_not(block_zero)

            @pl.when(block_zero)
            def _():
                drain_out(q, sout[p])
                for b in range(B):
                    def row(r, carry):
                        def col(c, carry2):
                            xv = x_v[p, b, r, pl.ds(c * L, L)]
                            pv = pe_v[q, r, pl.ds(c * L, L)]
                            o_v[r, pl.ds(c * L, L)] = jnp.where(xv == 0.0, 0.0, pv)
                            return carry2

                        return lax.fori_loop(0, ncol, col, carry)

                    lax.fori_loop(0, _R, row, None)
                    pltpu.sync_copy(o_v, out_hbm.at[b, pl.ds(s0, _R)])

            return fast

        def step(j, carry):
            fA, fB = carry
            flags = [fA, fB]
            for p_ in range(4):
                i = 4 * j + p_
                p = p_ % 2
                fast = body(i, p, p_, flags[p_])
                flags.append(fast)
                if p_ < 2:
                    fire_in(i + 2, p, (p_ + 2) % 4)
                else:

                    @pl.when(j < nblk // 4 - 1)
                    def _():
                        fire_in(i + 2, p, (p_ + 2) % 4)

            return flags[4], flags[5]

        fire_in(0, 0, 0)
        fire_in(1, 1, 1)
        f = jnp.bool_(False)
        fA, fB = lax.fori_loop(0, nblk // 4, step, (f, f))

        @pl.when(fA)
        def _():
            drain_out((nblk - 2) % 4, sout[0])

        @pl.when(fB)
        def _():
            drain_out((nblk - 1) % 4, sout[1])

    return k


def kernel(x, position_enc):
    B, S, D = x.shape
    pe = position_enc[:S]
    return _sc_kernel(B, S, D)(x, pe)
